# baseline (speedup 1.0000x reference)
import functools

import jax
import jax.numpy as jnp
from jax import lax
from jax.experimental import pallas as pl
from jax.experimental.pallas import tpu as pltpu
from jax.experimental.pallas import tpu_sc as plsc

NUM_FEATURES = 2
B = 2
S = 128
D = 64
REPS = 4
NW = 32
ROWS_PER_W = S // NW


def _sc_kernel_body(idx_hbm, kt0, vt0, kt1, vt1, k_out, v_out,
                    idx_v, k_rows, v_rows, gsem, ssem):
    wid = lax.axis_index("s") * 2 + lax.axis_index("c")
    i0 = wid * ROWS_PER_W
    k_tables = (kt0, kt1)
    v_tables = (vt0, vt1)
    pending = []
    for f in range(NUM_FEATURES):
        for b in range(B):
            for d in pending:
                d.wait()
            pending = []
            pltpu.sync_copy(idx_hbm.at[f, b, pl.ds(i0, ROWS_PER_W), :], idx_v)
            gathers = []
            for c in range(ROWS_PER_W):
                gathers.append(pltpu.async_copy(
                    k_tables[f].at[idx_v.at[c]], k_rows.at[c], gsem))
                gathers.append(pltpu.async_copy(
                    v_tables[f].at[idx_v.at[c]], v_rows.at[c], gsem))
            for g in gathers:
                g.wait()
            for r in range(REPS):
                h = REPS * f + r
                pending.append(pltpu.async_copy(
                    k_rows, k_out.at[b, h, pl.ds(i0, ROWS_PER_W), :, :], ssem))
                pending.append(pltpu.async_copy(
                    v_rows, v_out.at[b, h, pl.ds(i0, ROWS_PER_W), :, :], ssem))
    for d in pending:
        d.wait()


@jax.jit
def _tree_rel_pos(idx, kt0, vt0, kt1, vt1):
    out_sds = jax.ShapeDtypeStruct((B, NUM_FEATURES * REPS, S, S, D),
                                   jnp.float32)
    mesh = plsc.VectorSubcoreMesh(core_axis_name="c", subcore_axis_name="s")
    run = functools.partial(
        pl.kernel,
        out_type=[out_sds, out_sds],
        mesh=mesh,
        scratch_types=[
            pltpu.VMEM((ROWS_PER_W, S), jnp.int32),
            pltpu.VMEM((ROWS_PER_W, S, D), jnp.float32),
            pltpu.VMEM((ROWS_PER_W, S, D), jnp.float32),
            pltpu.SemaphoreType.DMA,
            pltpu.SemaphoreType.DMA,
        ],
        compiler_params=pltpu.CompilerParams(use_tc_tiling_on_sc=False),
    )(_sc_kernel_body)
    return run(idx, kt0, vt0, kt1, vt1)


def kernel(inputs, emb0, emb1):
    scale = float(D) ** 0.5
    kt0 = (emb0[:, :D] * scale)
    vt0 = (emb0[:, D:] * scale)
    kt1 = (emb1[:, :D] * scale)
    vt1 = (emb1[:, D:] * scale)
    k_out, v_out = _tree_rel_pos(inputs, kt0, vt0, kt1, vt1)
    return (k_out, v_out)


# speedup vs baseline: 1.1023x; 1.1023x over previous
import functools

import jax
import jax.numpy as jnp
from jax import lax
from jax.experimental import pallas as pl
from jax.experimental.pallas import tpu as pltpu
from jax.experimental.pallas import tpu_sc as plsc

NUM_FEATURES = 2
B = 2
S = 128
D = 64
REPS = 4
H = NUM_FEATURES * REPS
NW = 32
ROWS_PER_W = S // NW
CHUNK = 2
NBUF = 2


def _sc_gather_body(idx_hbm, kv0, kv1, inter, idx_v, buf0, buf1,
                    gsem0, gsem1, ssem0, ssem1):
    wid = lax.axis_index("s") * 2 + lax.axis_index("c")
    bufs = (buf0, buf1)
    gsems = (gsem0, gsem1)
    ssems = (ssem0, ssem1)
    pltpu.sync_copy(idx_hbm.at[wid], idx_v)
    tables = (kv0, kv1)
    units = []
    for f in range(NUM_FEATURES):
        for b in range(B):
            for half in range(ROWS_PER_W // CHUNK):
                units.append((f, b, half))
    scatters = [None] * NBUF
    gathers = [None] * NBUF

    def issue_gathers(t):
        f, b, half = units[t]
        slot = t % NBUF
        gs = []
        for c in range(CHUNK):
            row = (f * B + b) * ROWS_PER_W + half * CHUNK + c
            gs.append(pltpu.async_copy(
                tables[f].at[idx_v.at[row]], bufs[slot].at[c], gsems[slot]))
        gathers[slot] = gs

    issue_gathers(0)
    for t in range(len(units)):
        f, b, half = units[t]
        slot = t % NBUF
        for g in gathers[slot]:
            g.wait()
        if t + 1 < len(units):
            nslot = (t + 1) % NBUF
            if scatters[nslot] is not None:
                scatters[nslot].wait()
                scatters[nslot] = None
            issue_gathers(t + 1)
        i0 = wid * ROWS_PER_W + half * CHUNK
        scatters[slot] = pltpu.async_copy(
            bufs[slot], inter.at[f, b, pl.ds(i0, CHUNK), :, :], ssems[slot])
    for s in scatters:
        if s is not None:
            s.wait()


def _tc_replicate_body(inter_ref, k_ref, v_ref):
    for f in range(NUM_FEATURES):
        x = inter_ref[f, 0]
        k = x[:, :, :D]
        v = x[:, :, D:]
        for r in range(REPS):
            h = REPS * f + r
            k_ref[0, h] = k
            v_ref[0, h] = v


@jax.jit
def _tree_rel_pos(idx_perm, kv0, kv1):
    mesh = plsc.VectorSubcoreMesh(core_axis_name="c", subcore_axis_name="s")
    inter_sds = jax.ShapeDtypeStruct((NUM_FEATURES, B, S, S, 2 * D),
                                     jnp.float32)
    sc_run = functools.partial(
        pl.kernel,
        out_type=inter_sds,
        mesh=mesh,
        scratch_types=[
            pltpu.VMEM((NUM_FEATURES * B * ROWS_PER_W, S), jnp.int32),
            pltpu.VMEM((CHUNK, S, 2 * D), jnp.float32),
            pltpu.VMEM((CHUNK, S, 2 * D), jnp.float32),
            pltpu.SemaphoreType.DMA,
            pltpu.SemaphoreType.DMA,
            pltpu.SemaphoreType.DMA,
            pltpu.SemaphoreType.DMA,
        ],
    )(_sc_gather_body)
    inter = sc_run(idx_perm, kv0, kv1)

    blk = 8
    out_sds = jax.ShapeDtypeStruct((B, H, S, S, D), jnp.float32)
    k_out, v_out = pl.pallas_call(
        _tc_replicate_body,
        grid=(B, S // blk),
        in_specs=[pl.BlockSpec((NUM_FEATURES, 1, blk, S, 2 * D),
                               lambda b, i: (0, b, i, 0, 0))],
        out_specs=[
            pl.BlockSpec((1, H, blk, S, D), lambda b, i: (b, 0, i, 0, 0)),
            pl.BlockSpec((1, H, blk, S, D), lambda b, i: (b, 0, i, 0, 0)),
        ],
        out_shape=[out_sds, out_sds],
    )(inter)
    return k_out, v_out


def kernel(inputs, emb0, emb1):
    scale = float(D) ** 0.5
    idx_perm = jnp.transpose(
        inputs.reshape(NUM_FEATURES, B, NW, ROWS_PER_W, S),
        (2, 0, 1, 3, 4)).reshape(NW, NUM_FEATURES * B * ROWS_PER_W, S)
    return _tree_rel_pos(idx_perm, emb0 * scale, emb1 * scale)
